# baseline (speedup 1.0000x reference)
import functools

import jax
import jax.numpy as jnp
from jax import lax
from jax.experimental import pallas as pl
from jax.experimental.pallas import tpu as pltpu
from jax.experimental.pallas import tpu_sc as plsc

_D = 300
_E = 64
_NC = 2
_NS = 16
_NW = _NC * _NS
_BN = 4096
_NBLK = 123
_OFF = (_NBLK - 1) * _BN
_BB = 256


def _tc_project_table(tt, w):
    def mm(x1_ref, x2_ref, w_ref, o_ref):
        ww = w_ref[...]
        a1 = lax.dot_general(ww, x1_ref[...], (((1,), (0,)), ((), ())),
                             preferred_element_type=jnp.float32)
        a2 = lax.dot_general(ww, x2_ref[...], (((1,), (0,)), ((), ())),
                             preferred_element_type=jnp.float32)
        o_ref[...] = jnp.concatenate([a1.T, a2.T], axis=1)

    return pl.pallas_call(
        mm,
        grid=(_NBLK,),
        in_specs=[
            pl.BlockSpec((_D, _BN), lambda i: (0, i)),
            pl.BlockSpec((_D, _BN), lambda i: (0, i + _NBLK - 1)),
            pl.BlockSpec((_E, _D), lambda i: (0, 0)),
        ],
        out_specs=pl.BlockSpec((_BN, 2 * _E), lambda i: (i, 0)),
        out_shape=jax.ShapeDtypeStruct((_NBLK * _BN, 2 * _E), jnp.float32),
    )(tt, tt, w)


def _sc_gather(packed, idx):
    rows = idx.shape[0]
    bpw = rows // _NW
    fchunk = 160
    nchunk = bpw // fchunk
    mesh = plsc.VectorSubcoreMesh(core_axis_name="c", subcore_axis_name="s")

    @functools.partial(
        pl.kernel,
        mesh=mesh,
        compiler_params=pltpu.CompilerParams(needs_layout_passes=False),
        out_type=jax.ShapeDtypeStruct((rows * _E,), jnp.float32),
        scratch_types=[
            pltpu.VMEM((bpw,), jnp.int32),
            pltpu.VMEM((fchunk * 2 * _E,), jnp.float32),
            pltpu.VMEM((fchunk * 2 * _E,), jnp.float32),
            pltpu.VMEM((fchunk * _E,), jnp.float32),
            pltpu.VMEM((fchunk * _E,), jnp.float32),
            pltpu.SemaphoreType.DMA,
            pltpu.SemaphoreType.DMA,
            pltpu.SemaphoreType.DMA,
        ],
    )
    def gather_kernel(idx_hbm, tab_hbm, out_hbm, idx_v, buf0, buf1,
                      obuf0, obuf1, sem_g, sem_o0, sem_o1):
        wid = lax.axis_index("s") * _NC + lax.axis_index("c")
        fbase = pl.multiple_of(wid * bpw, fchunk)
        pltpu.sync_copy(idx_hbm.at[pl.ds(fbase, bpw)], idx_v)
        bufs = (buf0, buf1)
        obufs = (obuf0, obuf1)
        sem_os = (sem_o0, sem_o1)

        def fire(c, buf):
            def body(g, _):
                off = pl.multiple_of(c * fchunk + g * 16, 16)
                vec = idx_v[pl.ds(off, 16)]
                vec = (vec - jnp.where(vec >= _OFF, _OFF, 0)) * (2 * _E)
                for e in range(16):
                    pltpu.async_copy(
                        tab_hbm.at[pl.ds(
                            pl.multiple_of(vec[e], 2 * _E), 2 * _E)],
                        buf.at[pl.ds(
                            pl.multiple_of((g * 16 + e) * 2 * _E, 2 * _E),
                            2 * _E)], sem_g)
                return 0
            lax.fori_loop(0, fchunk // 16, body, 0)

        def pick(c, buf, obuf):
            def pbody(g, _):
                off = pl.multiple_of(c * fchunk + g * 16, 16)
                vs = idx_v[pl.ds(off, 16)]
                h = jnp.where(vs >= _OFF, _E, 0)
                jvec = lax.iota(jnp.int32, 16) + g * 16
                src = jvec * (2 * _E) + h
                dst = jvec * _E
                for m in range(_E):
                    val = plsc.load_gather(buf, [src + m])
                    plsc.store_scatter(obuf, [dst + m], val)
                return 0
            lax.fori_loop(0, fchunk // 16, pbody, 0)

        fire(0, buf0)

        def outer(i, _):
            for b2 in range(2):
                buf = bufs[b2]
                obuf = obufs[b2]
                sem_ob = sem_os[b2]
                c = i * 2 + b2
                pltpu.make_async_copy(
                    tab_hbm.at[pl.ds(0, fchunk * 2 * _E)], buf, sem_g).wait()
                pl.when(c + 1 < nchunk)(
                    lambda: fire(c + 1, bufs[1 - b2]))
                pl.when(i > 0)(
                    lambda: pltpu.make_async_copy(
                        tab_hbm.at[pl.ds(0, fchunk * _E)], obuf,
                        sem_ob).wait())
                pick(c, buf, obuf)
                pltpu.async_copy(
                    obuf, out_hbm.at[pl.ds(
                        pl.multiple_of((fbase + c * fchunk) * _E,
                                       fchunk * _E), fchunk * _E)],
                    sem_ob)
            return 0

        lax.fori_loop(0, nchunk // 2, outer, 0)
        for b2 in range(2):
            pltpu.make_async_copy(
                tab_hbm.at[pl.ds(0, fchunk * _E)], obufs[b2],
                sem_os[b2]).wait()

    return gather_kernel(idx, packed.reshape(-1))


def kernel(inputs, table, W):
    b, l = inputs.shape
    idx = inputs.reshape(-1)
    packed = _tc_project_table(table.T, W)
    out = _sc_gather(packed, idx)
    return out.reshape(b, l, _E)


# speedup vs baseline: 1.2120x; 1.2120x over previous
import functools

import jax
import jax.numpy as jnp
from jax import lax
from jax.experimental import pallas as pl
from jax.experimental.pallas import tpu as pltpu
from jax.experimental.pallas import tpu_sc as plsc

_D = 300
_E = 64
_NC = 2
_NS = 16
_NW = _NC * _NS
_BN = 8192
_NBLK = 62
_OFF = (_NBLK - 1) * _BN
_BB = 256


def _tc_project_table(tt, w):
    def mm(x1_ref, x2_ref, w_ref, o_ref):
        ww = w_ref[...]
        a1 = lax.dot_general(ww, x1_ref[...], (((1,), (0,)), ((), ())),
                             preferred_element_type=jnp.float32)
        a2 = lax.dot_general(ww, x2_ref[...], (((1,), (0,)), ((), ())),
                             preferred_element_type=jnp.float32)
        o_ref[...] = jnp.concatenate([a1.T, a2.T], axis=1)

    return pl.pallas_call(
        mm,
        grid=(_NBLK,),
        in_specs=[
            pl.BlockSpec((_D, _BN), lambda i: (0, i)),
            pl.BlockSpec((_D, _BN), lambda i: (0, i + _NBLK - 1)),
            pl.BlockSpec((_E, _D), lambda i: (0, 0)),
        ],
        out_specs=pl.BlockSpec((_BN, 2 * _E), lambda i: (i, 0)),
        out_shape=jax.ShapeDtypeStruct((_NBLK * _BN, 2 * _E), jnp.float32),
    )(tt, tt, w)


def _sc_gather(packed, idx, b, l):
    rows = idx.shape[0]
    bpw = rows // _NW
    bb_pw = b // _NW
    bchunk = 8
    fchunk = bchunk * l
    nchunk = bb_pw // bchunk
    mesh = plsc.VectorSubcoreMesh(core_axis_name="c", subcore_axis_name="s")

    @functools.partial(
        pl.kernel,
        mesh=mesh,
        out_type=jax.ShapeDtypeStruct((b, l, 2 * _E), jnp.float32),
        scratch_types=[
            pltpu.VMEM((bpw,), jnp.int32),
            pltpu.VMEM((fchunk, 2 * _E), jnp.float32),
            pltpu.VMEM((fchunk, 2 * _E), jnp.float32),
            pltpu.SemaphoreType.DMA,
            pltpu.SemaphoreType.DMA,
        ],
    )
    def gather_kernel(idx_hbm, tab_hbm, out_hbm, idx_v, buf0, buf1,
                      sem_g, sem_o):
        wid = lax.axis_index("s") * _NC + lax.axis_index("c")
        fbase = wid * bpw
        bbase = wid * bb_pw
        pltpu.sync_copy(idx_hbm.at[pl.ds(fbase, bpw)], idx_v)
        bufs = (buf0, buf1)
        for c in range(nchunk):
            buf = bufs[c % 2]
            if c >= 2:
                pltpu.make_async_copy(
                    tab_hbm.at[pl.ds(0, fchunk)], buf, sem_o).wait()

            def body(g, _, c=c, buf=buf):
                vec = idx_v[pl.ds(c * fchunk + g * 16, 16)]
                vec = vec - jnp.where(vec >= _OFF, _OFF, 0)
                for e in range(16):
                    pltpu.async_copy(
                        tab_hbm.at[pl.ds(vec[e], 1)],
                        buf.at[pl.ds(g * 16 + e, 1)], sem_g)
                return 0
            lax.fori_loop(0, fchunk // 16, body, 0)
            pltpu.make_async_copy(
                tab_hbm.at[pl.ds(0, fchunk)], buf, sem_g).wait()
            for k in range(bchunk):
                pltpu.async_copy(
                    buf.at[pl.ds(k * l, l)],
                    out_hbm.at[bbase + c * bchunk + k], sem_o)
        for tail in range(min(2, nchunk)):
            pltpu.make_async_copy(
                tab_hbm.at[pl.ds(0, fchunk)], bufs[tail], sem_o).wait()

    return gather_kernel(idx, packed)


def _tc_select(g3, par3):
    b, l, _ = g3.shape

    def sel(g_ref, p_ref, o_ref):
        gg = g_ref[...]
        o_ref[...] = jnp.where(p_ref[...] != 0, gg[:, :, _E:], gg[:, :, :_E])

    return pl.pallas_call(
        sel,
        grid=(b // _BB,),
        in_specs=[
            pl.BlockSpec((_BB, l, 2 * _E), lambda i: (i, 0, 0)),
            pl.BlockSpec((_BB, l, _E), lambda i: (i, 0, 0)),
        ],
        out_specs=pl.BlockSpec((_BB, l, _E), lambda i: (i, 0, 0)),
        out_shape=jax.ShapeDtypeStruct((b, l, _E), jnp.float32),
    )(g3, par3)


def kernel(inputs, table, W):
    b, l = inputs.shape
    idx = inputs.reshape(-1)
    packed = _tc_project_table(table.T, W)
    g3 = _sc_gather(packed, idx, b, l)
    par3 = jnp.broadcast_to(
        (inputs >= _OFF).astype(jnp.int8)[:, :, None], (b, l, _E))
    return _tc_select(g3, par3)


# speedup vs baseline: 1.3089x; 1.0799x over previous
import functools

import jax
import jax.numpy as jnp
from jax import lax
from jax.experimental import pallas as pl
from jax.experimental.pallas import tpu as pltpu
from jax.experimental.pallas import tpu_sc as plsc

_D = 300
_E = 64
_NC = 2
_NS = 16
_NW = _NC * _NS
_BN = 4096
_NBLK = 62
_OFF = (_NBLK - 1) * _BN
_BB = 256


def _tc_project_table(tt, w):
    def rne16(x):
        bits = lax.bitcast_convert_type(x, jnp.uint32)
        return (bits + 0x7FFF + ((bits >> 16) & 1)) >> 16

    def mm(x0_ref, x1_ref, x2_ref, x3_ref, w_ref, o_ref):
        ww = w_ref[...]
        dn = (((1,), (0,)), ((), ()))
        a = [lax.dot_general(ww, x_ref[...], dn,
                             preferred_element_type=jnp.float32).T
             for x_ref in (x0_ref, x1_ref, x2_ref, x3_ref)]
        lo = rne16(a[0]) | (rne16(a[1]) << 16)
        hi = rne16(a[2]) | (rne16(a[3]) << 16)
        o_ref[...] = lax.bitcast_convert_type(
            jnp.concatenate([lo, hi], axis=1), jnp.float32)

    return pl.pallas_call(
        mm,
        grid=(_NBLK,),
        in_specs=[
            pl.BlockSpec((_D, _BN), lambda i: (0, i)),
            pl.BlockSpec((_D, _BN), lambda i: (0, i + (_NBLK - 1))),
            pl.BlockSpec((_D, _BN), lambda i: (0, i + 2 * (_NBLK - 1))),
            pl.BlockSpec((_D, _BN), lambda i: (0, i + 3 * (_NBLK - 1))),
            pl.BlockSpec((_E, _D), lambda i: (0, 0)),
        ],
        out_specs=pl.BlockSpec((_BN, 2 * _E), lambda i: (i, 0)),
        out_shape=jax.ShapeDtypeStruct((_NBLK * _BN, 2 * _E), jnp.float32),
    )(tt, tt, tt, tt, w)


def _sc_gather(packed, idx, b, l):
    rows = idx.shape[0]
    bpw = rows // _NW
    bb_pw = b // _NW
    bchunk = 8
    fchunk = bchunk * l
    nchunk = bb_pw // bchunk
    mesh = plsc.VectorSubcoreMesh(core_axis_name="c", subcore_axis_name="s")

    @functools.partial(
        pl.kernel,
        mesh=mesh,
        out_type=jax.ShapeDtypeStruct((b, l, 2 * _E), jnp.float32),
        scratch_types=[
            pltpu.VMEM((bpw,), jnp.int32),
            pltpu.VMEM((fchunk, 2 * _E), jnp.float32),
            pltpu.VMEM((fchunk, 2 * _E), jnp.float32),
            pltpu.SemaphoreType.DMA,
            pltpu.SemaphoreType.DMA,
        ],
    )
    def gather_kernel(idx_hbm, tab_hbm, out_hbm, idx_v, buf0, buf1,
                      sem_g, sem_o):
        wid = lax.axis_index("s") * _NC + lax.axis_index("c")
        fbase = wid * bpw
        bbase = wid * bb_pw
        pltpu.sync_copy(idx_hbm.at[pl.ds(fbase, bpw)], idx_v)
        bufs = (buf0, buf1)
        for c in range(nchunk):
            buf = bufs[c % 2]
            if c >= 2:
                pltpu.make_async_copy(
                    tab_hbm.at[pl.ds(0, fchunk)], buf, sem_o).wait()

            def body(g, _, c=c, buf=buf):
                vec = idx_v[pl.ds(c * fchunk + g * 16, 16)]
                vec = (vec
                       - jnp.where(vec >= _OFF, _OFF, 0)
                       - jnp.where(vec >= 2 * _OFF, _OFF, 0)
                       - jnp.where(vec >= 3 * _OFF, _OFF, 0))
                for e in range(16):
                    pltpu.async_copy(
                        tab_hbm.at[pl.ds(vec[e], 1)],
                        buf.at[pl.ds(g * 16 + e, 1)], sem_g)
                return 0
            lax.fori_loop(0, fchunk // 16, body, 0)
            pltpu.make_async_copy(
                tab_hbm.at[pl.ds(0, fchunk)], buf, sem_g).wait()
            for k in range(bchunk):
                pltpu.async_copy(
                    buf.at[pl.ds(k * l, l)],
                    out_hbm.at[bbase + c * bchunk + k], sem_o)
        for tail in range(min(2, nchunk)):
            pltpu.make_async_copy(
                tab_hbm.at[pl.ds(0, fchunk)], bufs[tail], sem_o).wait()

    return gather_kernel(idx, packed)


def _tc_select(g3, q3):
    b, l, _ = g3.shape

    def sel(g_ref, q_ref, o_ref):
        q32 = q_ref[...].astype(jnp.int32)
        u = lax.bitcast_convert_type(g_ref[...], jnp.int32)
        uh = jnp.where(q32 >= 2, u[:, :, _E:], u[:, :, :_E])
        lo = lax.bitcast_convert_type(uh << 16, jnp.float32)
        hi = lax.bitcast_convert_type(
            uh & jnp.int32(-65536), jnp.float32)
        o_ref[...] = jnp.where((q32 & 1) == 1, hi, lo)

    return pl.pallas_call(
        sel,
        grid=(b // _BB,),
        in_specs=[
            pl.BlockSpec((_BB, l, 2 * _E), lambda i: (i, 0, 0)),
            pl.BlockSpec((_BB, l, _E), lambda i: (i, 0, 0)),
        ],
        out_specs=pl.BlockSpec((_BB, l, _E), lambda i: (i, 0, 0)),
        out_shape=jax.ShapeDtypeStruct((b, l, _E), jnp.float32),
    )(g3, q3)


def kernel(inputs, table, W):
    b, l = inputs.shape
    idx = inputs.reshape(-1)
    packed = _tc_project_table(table.T, W)
    g3 = _sc_gather(packed, idx, b, l)
    q = ((inputs >= _OFF).astype(jnp.int8)
         + (inputs >= 2 * _OFF).astype(jnp.int8)
         + (inputs >= 3 * _OFF).astype(jnp.int8))
    q3 = jnp.broadcast_to(q[:, :, None], (b, l, _E))
    return _tc_select(g3, q3)


# speedup vs baseline: 1.3733x; 1.0492x over previous
import functools

import jax
import jax.numpy as jnp
from jax import lax
from jax.experimental import pallas as pl
from jax.experimental.pallas import tpu as pltpu
from jax.experimental.pallas import tpu_sc as plsc

_D = 300
_E = 64
_NC = 2
_NS = 16
_NW = _NC * _NS
_BN = 4096
_NBLK = 62
_OFF = (_NBLK - 1) * _BN
_BB = 256


def _tc_project_table(tt, w):
    def rne16(x):
        bits = lax.bitcast_convert_type(x, jnp.uint32)
        return (bits + 0x7FFF + ((bits >> 16) & 1)) >> 16

    def mm(x0_ref, x1_ref, x2_ref, x3_ref, w_ref, o_ref):
        ww = w_ref[...]
        dn = (((1,), (0,)), ((), ()))
        a = [lax.dot_general(ww, x_ref[...], dn,
                             preferred_element_type=jnp.float32).T
             for x_ref in (x0_ref, x1_ref, x2_ref, x3_ref)]
        lo = rne16(a[0]) | (rne16(a[1]) << 16)
        hi = rne16(a[2]) | (rne16(a[3]) << 16)
        o_ref[...] = lax.bitcast_convert_type(
            jnp.concatenate([lo, hi], axis=1), jnp.float32)

    return pl.pallas_call(
        mm,
        grid=(_NBLK,),
        in_specs=[
            pl.BlockSpec((_D, _BN), lambda i: (0, i)),
            pl.BlockSpec((_D, _BN), lambda i: (0, i + (_NBLK - 1))),
            pl.BlockSpec((_D, _BN), lambda i: (0, i + 2 * (_NBLK - 1))),
            pl.BlockSpec((_D, _BN), lambda i: (0, i + 3 * (_NBLK - 1))),
            pl.BlockSpec((_E, _D), lambda i: (0, 0)),
        ],
        out_specs=pl.BlockSpec((_BN, 2 * _E), lambda i: (i, 0)),
        out_shape=jax.ShapeDtypeStruct((_NBLK * _BN, 2 * _E), jnp.float32),
    )(tt, tt, tt, tt, w)


def _sc_gather(packed, idx):
    rows = idx.shape[0]
    bpw = rows // _NW
    fchunk = 160
    nchunk = bpw // fchunk
    mesh = plsc.VectorSubcoreMesh(core_axis_name="c", subcore_axis_name="s")

    @functools.partial(
        pl.kernel,
        mesh=mesh,
        out_type=jax.ShapeDtypeStruct((rows * _E,), jnp.float32),
        scratch_types=[
            pltpu.VMEM((bpw,), jnp.int32),
            pltpu.VMEM((fchunk * _E,), jnp.float32),
            pltpu.VMEM((fchunk * _E,), jnp.float32),
            pltpu.SemaphoreType.DMA,
            pltpu.SemaphoreType.DMA,
        ],
    )
    def gather_kernel(idx_hbm, tab_hbm, out_hbm, idx_v, buf0, buf1,
                      sem_g, sem_o):
        wid = lax.axis_index("s") * _NC + lax.axis_index("c")
        fbase = pl.multiple_of(wid * bpw, fchunk)
        pltpu.sync_copy(idx_hbm.at[pl.ds(fbase, bpw)], idx_v)
        bufs = (buf0, buf1)
        for c in range(nchunk):
            buf = bufs[c % 2]
            if c >= 2:
                pltpu.make_async_copy(
                    tab_hbm.at[pl.ds(0, fchunk * _E)], buf, sem_o).wait()

            def body(g, _, c=c, buf=buf):
                off = pl.multiple_of(c * fchunk + g * 16, 16)
                vec = idx_v[pl.ds(off, 16)]
                row = (vec
                       - jnp.where(vec >= _OFF, _OFF, 0)
                       - jnp.where(vec >= 2 * _OFF, _OFF, 0)
                       - jnp.where(vec >= 3 * _OFF, _OFF, 0))
                src = row * (2 * _E) + jnp.where(vec >= 2 * _OFF, _E, 0)
                for e in range(16):
                    pltpu.async_copy(
                        tab_hbm.at[pl.ds(
                            pl.multiple_of(src[e], _E), _E)],
                        buf.at[pl.ds(
                            pl.multiple_of((g * 16 + e) * _E, _E), _E)],
                        sem_g)
                return 0
            lax.fori_loop(0, fchunk // 16, body, 0)
            pltpu.make_async_copy(
                tab_hbm.at[pl.ds(0, fchunk * _E)], buf, sem_g).wait()
            pltpu.async_copy(
                buf, out_hbm.at[pl.ds(
                    pl.multiple_of((fbase + c * fchunk) * _E, fchunk * _E),
                    fchunk * _E)], sem_o)
        for tail in range(min(2, nchunk)):
            pltpu.make_async_copy(
                tab_hbm.at[pl.ds(0, fchunk * _E)], bufs[tail], sem_o).wait()

    return gather_kernel(idx, packed.reshape(-1))


def _tc_select(g2, par):
    n = g2.shape[0]
    br = 2560

    def sel(g_ref, q_ref, o_ref):
        q32 = q_ref[...].astype(jnp.int32)
        u = lax.bitcast_convert_type(g_ref[...], jnp.int32)
        lo = lax.bitcast_convert_type(u << 16, jnp.float32)
        hi = lax.bitcast_convert_type(u & jnp.int32(-65536), jnp.float32)
        o_ref[...] = jnp.where(q32 == 1, hi, lo)

    return pl.pallas_call(
        sel,
        grid=(n // br,),
        in_specs=[
            pl.BlockSpec((br, 2 * _E), lambda i: (i, 0)),
            pl.BlockSpec((br, 2 * _E), lambda i: (i, 0)),
        ],
        out_specs=pl.BlockSpec((br, 2 * _E), lambda i: (i, 0)),
        out_shape=jax.ShapeDtypeStruct((n, 2 * _E), jnp.float32),
    )(g2, par)


def kernel(inputs, table, W):
    b, l = inputs.shape
    rows = b * l
    idx = inputs.reshape(-1)
    packed = _tc_project_table(table.T, W)
    gf = _sc_gather(packed, idx)
    g2 = gf.reshape(rows // 2, 2 * _E)
    odd = ((inputs >= _OFF).astype(jnp.int8)
           ^ (inputs >= 2 * _OFF).astype(jnp.int8)
           ^ (inputs >= 3 * _OFF).astype(jnp.int8))
    par = jnp.repeat(odd.reshape(rows // 2, 2), _E, axis=1)
    out = _tc_select(g2, par)
    return out.reshape(b, l, _E)
